# baseline (speedup 1.0000x reference)
import functools

import jax
import jax.numpy as jnp
from jax import lax
from jax.experimental import pallas as pl
from jax.experimental.pallas import tpu as pltpu
from jax.experimental.pallas import tpu_sc as plsc

NC = 2
NS = 16
L = 16


@functools.lru_cache(maxsize=None)
def _make_kernel(B, N, D):
    NW = NC * NS
    bpw = B // NW
    CH = 128
    nch = bpw // CH
    ng = bpw // L

    mesh = plsc.VectorSubcoreMesh(core_axis_name="c", subcore_axis_name="s")

    @functools.partial(
        pl.kernel,
        mesh=mesh,
        compiler_params=pltpu.CompilerParams(
            needs_layout_passes=False, use_tc_tiling_on_sc=False),
        out_type=jax.ShapeDtypeStruct((B,), jnp.float32),
        scratch_types=[
            pltpu.VMEM((bpw,), jnp.int32),
            pltpu.VMEM((bpw,), jnp.int32),
            pltpu.VMEM((D, bpw), jnp.float32),
            pltpu.VMEM((D, bpw), jnp.float32),
            pltpu.VMEM((bpw,), jnp.float32),
            pltpu.VMEM((L,), jnp.float32),
            pltpu.VMEM((L,), jnp.float32),
            pltpu.SemaphoreType.DMA,
        ],
    )
    def k(ui_hbm, mi_hbm, ut_hbm, mt_hbm, w_hbm, b_hbm, out_hbm,
          ui_v, mi_v, up_v, mp_v, out_v, w_v, b_v, sem):
        wid = lax.axis_index("s") * NC + lax.axis_index("c")
        base = wid * bpw
        pltpu.sync_copy(ui_hbm.at[pl.ds(base, bpw)], ui_v)
        pltpu.sync_copy(mi_hbm.at[pl.ds(base, bpw)], mi_v)
        pltpu.sync_copy(w_hbm, w_v)
        pltpu.sync_copy(b_hbm, b_v)

        copies = []
        for d in range(D):
            for c in range(nch):
                sl = pl.ds(c * CH, CH)
                copies.append(pltpu.async_copy(
                    ut_hbm.at[d].at[ui_v.at[sl]], up_v.at[d].at[sl], sem))
                copies.append(pltpu.async_copy(
                    mt_hbm.at[d].at[mi_v.at[sl]], mp_v.at[d].at[sl], sem))
        for cp in copies:
            cp.wait()

        w = w_v[...]
        b = b_v[...]

        def grp(g, carry):
            sl = pl.ds(g * L, L)
            acc = jnp.zeros((L,), jnp.float32)
            for d in range(D):
                acc = acc + up_v[d, sl] * mp_v[d, sl]
            out_v[sl] = acc * w + b
            return carry
        lax.fori_loop(0, ng, grp, 0)

        pltpu.sync_copy(out_v, out_hbm.at[pl.ds(base, bpw)])

    return k


@jax.jit
def kernel(inputs, user_table, movie_table, dense_w, dense_b):
    B = inputs.shape[0]
    N, D = user_table.shape
    idx = inputs.astype(jnp.int32)
    out = _make_kernel(B, N, D)(
        idx[:, 0], idx[:, 1], user_table.T, movie_table.T,
        jnp.full((L,), dense_w[0, 0], jnp.float32),
        jnp.full((L,), dense_b[0], jnp.float32),
    )
    return out.reshape(B, 1)


# speedup vs baseline: 7.5130x; 7.5130x over previous
import functools

import jax
import jax.numpy as jnp
from jax import lax
from jax.experimental import pallas as pl
from jax.experimental.pallas import tpu as pltpu
from jax.experimental.pallas import tpu_sc as plsc

NC = 2
NS = 16
L = 16
SUB = 8


@functools.lru_cache(maxsize=None)
def _make_kernel(B, N, D):
    NW = NC * NS
    bpw = B // NW
    ng = bpw // L
    BLK = SUB * D
    GB = L * BLK

    mesh = plsc.VectorSubcoreMesh(core_axis_name="c", subcore_axis_name="s")

    @functools.partial(
        pl.kernel,
        mesh=mesh,
        compiler_params=pltpu.CompilerParams(needs_layout_passes=False),
        out_type=jax.ShapeDtypeStruct((B,), jnp.float32),
        scratch_types=[
            pltpu.VMEM((bpw,), jnp.int32),
            pltpu.VMEM((bpw,), jnp.int32),
            pltpu.VMEM((L, SUB, D), jnp.float32),
            pltpu.VMEM((L, SUB, D), jnp.float32),
            pltpu.VMEM((L, SUB, D), jnp.float32),
            pltpu.VMEM((L, SUB, D), jnp.float32),
            pltpu.VMEM((bpw,), jnp.float32),
            pltpu.VMEM((L,), jnp.float32),
            pltpu.VMEM((L,), jnp.float32),
            pltpu.SemaphoreType.DMA,
            pltpu.SemaphoreType.DMA,
        ],
    )
    def k(ui_hbm, mi_hbm, ut_hbm, mt_hbm, w_hbm, b_hbm, out_hbm,
          ui_v, mi_v, ub0, ub1, mb0, mb1, out_v, w_v, b_v, sem0, sem1):
        wid = lax.axis_index("s") * NC + lax.axis_index("c")
        base = wid * bpw
        pltpu.sync_copy(ui_hbm.at[pl.ds(base, bpw)], ui_v)
        pltpu.sync_copy(mi_hbm.at[pl.ds(base, bpw)], mi_v)
        pltpu.sync_copy(w_hbm, w_v)
        pltpu.sync_copy(b_hbm, b_v)

        ubufs = (ub0, ub1)
        mbufs = (mb0, mb1)
        sems = (sem0, sem1)

        def fire(g, parity):
            uvec = ui_v[pl.ds(g * L, L)]
            mvec = mi_v[pl.ds(g * L, L)]
            s = sems[parity]
            for i in range(L):
                u = uvec[i]
                m = mvec[i]
                pltpu.async_copy(ut_hbm.at[u >> 3], ubufs[parity].at[i], s)
                pltpu.async_copy(mt_hbm.at[m >> 3], mbufs[parity].at[i], s)

        def drain(parity):
            s = sems[parity]
            dummy = ut_hbm.at[pl.ds(0, L)]
            pltpu.make_async_copy(dummy, ubufs[parity], s).wait()
            pltpu.make_async_copy(dummy, mbufs[parity], s).wait()

        iota = lax.iota(jnp.int32, L)
        w = w_v[...]
        b = b_v[...]

        def compute(g, parity):
            ub, mb = ubufs[parity], mbufs[parity]
            uvec = ui_v[pl.ds(g * L, L)]
            mvec = mi_v[pl.ds(g * L, L)]
            acc = jnp.zeros((L,), jnp.float32)
            for i in range(L):
                s = jnp.sum(ub[i, uvec[i] & 7] * mb[i, mvec[i] & 7])
                acc = jnp.where(iota == i, s, acc)
            out_v[pl.ds(g * L, L)] = acc * w + b

        fire(0, 0)

        def step2(h, carry):
            g = h * 2
            fire(g + 1, 1)
            drain(0)
            compute(g, 0)
            fire(g + 2, 0)
            drain(1)
            compute(g + 1, 1)
            return carry
        lax.fori_loop(0, (ng - 2) // 2, step2, 0)

        g = ng - 2
        fire(g + 1, 1)
        drain(0)
        compute(g, 0)
        drain(1)
        compute(g + 1, 1)

        pltpu.sync_copy(out_v, out_hbm.at[pl.ds(base, bpw)])

    return k


@jax.jit
def kernel(inputs, user_table, movie_table, dense_w, dense_b):
    B = inputs.shape[0]
    N, D = user_table.shape
    idx = inputs.astype(jnp.int32)
    ut3 = user_table.reshape(N // SUB, SUB, D)
    mt3 = movie_table.reshape(N // SUB, SUB, D)
    out = _make_kernel(B, N, D)(
        idx[:, 0], idx[:, 1], ut3, mt3,
        jnp.full((L,), dense_w[0, 0], jnp.float32),
        jnp.full((L,), dense_b[0], jnp.float32),
    )
    return out.reshape(B, 1)
